# baseline (device time: 99805 ns/iter reference)
import jax
import jax.numpy as jnp
from jax import lax
from jax.experimental import pallas as pl
from jax.experimental.pallas import tpu as pltpu

Z = 4
M_BLK = 128
NC = 8
HALF = 512
CR = HALF // NC


def kernel(dy, W):
    m, k = dy.shape
    d = W.shape[0]
    nm = m // M_BLK

    def body(dy_ref, w_ref, out_ref, w_bf,
             sbuf_h1, sbuf_h2, rbufL_h1, rbufL_h2, rbufR_h1, rbufR_h2,
             fin_h1, fin_h2,
             rL_h1, rL_h2, rR_h1, rR_h2, finr_h1, finr_h2,
             s_red_h1, s_red_h2, s_finA_h1, s_finB_h1, s_finA_h2, s_finB_h2):
        i = pl.program_id(0)

        @pl.when(i == 0)
        def _():
            w_bf[...] = w_ref[...].astype(jnp.bfloat16)

        out_ref[pl.ds(i * M_BLK, M_BLK), :] = lax.dot_general(
            dy_ref[...].astype(jnp.bfloat16),
            w_bf[...],
            (((1,), (1,)), ((), ())),
            preferred_element_type=jnp.float32,
        )

        @pl.when(i == nm - 1)
        def _():
            my_x = lax.axis_index("x")
            my_y = lax.axis_index("y")
            my_z = lax.axis_index("z")

            def dev(zz):
                return dict(device_id=(my_x, my_y, zz),
                            device_id_type=pl.DeviceIdType.MESH)

            def copy(src, dst, ssem, rsem, zz):
                return pltpu.make_async_remote_copy(
                    src_ref=src, dst_ref=dst, send_sem=ssem, recv_sem=rsem,
                    **dev(zz))

            def rows(h, c):
                return pl.ds(h * HALF + c * CR, CR)

            bar = pltpu.get_barrier_semaphore()

            def barrier(nbrs):
                for nb in nbrs:
                    pl.semaphore_signal(bar, inc=1, **dev(nb))
                pl.semaphore_wait(bar, len(nbrs))

            @pl.when(my_z == 0)
            def _():
                barrier([1])
                sends = []
                for c in range(NC):
                    sbuf_h1[c] = out_ref[rows(0, c), :].astype(jnp.bfloat16)
                    r = copy(sbuf_h1.at[c], rbufL_h1.at[c],
                             s_red_h1.at[c], rL_h1.at[c], 1)
                    r.start(); sends.append(r)
                    sbuf_h2[c] = out_ref[rows(1, c), :].astype(jnp.bfloat16)
                    r = copy(sbuf_h2.at[c], rbufL_h2.at[c],
                             s_red_h2.at[c], rL_h2.at[c], 1)
                    r.start(); sends.append(r)
                for c in range(NC):
                    copy(fin_h1.at[c], fin_h1.at[c],
                         s_finA_h1.at[c], finr_h1.at[c], 1).wait_recv()
                    out_ref[rows(0, c), :] = fin_h1[c].astype(jnp.float32)
                    copy(fin_h2.at[c], fin_h2.at[c],
                         s_finA_h2.at[c], finr_h2.at[c], 1).wait_recv()
                    out_ref[rows(1, c), :] = fin_h2[c].astype(jnp.float32)
                for r in sends:
                    r.wait_send()

            @pl.when(my_z == 3)
            def _():
                barrier([2])
                sends = []
                for c in range(NC):
                    sbuf_h2[c] = out_ref[rows(1, c), :].astype(jnp.bfloat16)
                    r = copy(sbuf_h2.at[c], rbufR_h2.at[c],
                             s_red_h2.at[c], rR_h2.at[c], 2)
                    r.start(); sends.append(r)
                    sbuf_h1[c] = out_ref[rows(0, c), :].astype(jnp.bfloat16)
                    r = copy(sbuf_h1.at[c], rbufR_h1.at[c],
                             s_red_h1.at[c], rR_h1.at[c], 2)
                    r.start(); sends.append(r)
                for c in range(NC):
                    copy(fin_h2.at[c], fin_h2.at[c],
                         s_finA_h2.at[c], finr_h2.at[c], 2).wait_recv()
                    out_ref[rows(1, c), :] = fin_h2[c].astype(jnp.float32)
                    copy(fin_h1.at[c], fin_h1.at[c],
                         s_finA_h1.at[c], finr_h1.at[c], 2).wait_recv()
                    out_ref[rows(0, c), :] = fin_h1[c].astype(jnp.float32)
                for r in sends:
                    r.wait_send()

            @pl.when(my_z == 1)
            def _():
                barrier([0, 2])
                sends = []
                for c in range(NC):
                    copy(rbufL_h2.at[c], rbufL_h2.at[c],
                         s_red_h2.at[c], rL_h2.at[c], 0).wait_recv()
                    sbuf_h2[c] = (rbufL_h2[c].astype(jnp.float32)
                                  + out_ref[rows(1, c), :]).astype(jnp.bfloat16)
                    r = copy(sbuf_h2.at[c], rbufL_h2.at[c],
                             s_red_h2.at[c], rL_h2.at[c], 2)
                    r.start(); sends.append(r)
                    copy(rbufL_h1.at[c], rbufL_h1.at[c],
                         s_red_h1.at[c], rL_h1.at[c], 0).wait_recv()
                    copy(rbufR_h1.at[c], rbufR_h1.at[c],
                         s_red_h1.at[c], rR_h1.at[c], 2).wait_recv()
                    out_ref[rows(0, c), :] += (
                        rbufL_h1[c].astype(jnp.float32)
                        + rbufR_h1[c].astype(jnp.float32))
                    fin_h1[c] = out_ref[rows(0, c), :].astype(jnp.bfloat16)
                    r = copy(fin_h1.at[c], fin_h1.at[c],
                             s_finA_h1.at[c], finr_h1.at[c], 0)
                    r.start(); sends.append(r)
                    r = copy(fin_h1.at[c], fin_h1.at[c],
                             s_finB_h1.at[c], finr_h1.at[c], 2)
                    r.start(); sends.append(r)
                    copy(fin_h2.at[c], fin_h2.at[c],
                         s_finB_h2.at[c], finr_h2.at[c], 2).wait_recv()
                    out_ref[rows(1, c), :] = fin_h2[c].astype(jnp.float32)
                    r = copy(fin_h2.at[c], fin_h2.at[c],
                             s_finA_h2.at[c], finr_h2.at[c], 0)
                    r.start(); sends.append(r)
                for r in sends:
                    r.wait_send()

            @pl.when(my_z == 2)
            def _():
                barrier([1, 3])
                sends = []
                for c in range(NC):
                    copy(rbufR_h1.at[c], rbufR_h1.at[c],
                         s_red_h1.at[c], rR_h1.at[c], 3).wait_recv()
                    sbuf_h1[c] = (rbufR_h1[c].astype(jnp.float32)
                                  + out_ref[rows(0, c), :]).astype(jnp.bfloat16)
                    r = copy(sbuf_h1.at[c], rbufR_h1.at[c],
                             s_red_h1.at[c], rR_h1.at[c], 1)
                    r.start(); sends.append(r)
                    copy(rbufR_h2.at[c], rbufR_h2.at[c],
                         s_red_h2.at[c], rR_h2.at[c], 3).wait_recv()
                    copy(rbufL_h2.at[c], rbufL_h2.at[c],
                         s_red_h2.at[c], rL_h2.at[c], 1).wait_recv()
                    out_ref[rows(1, c), :] += (
                        rbufR_h2[c].astype(jnp.float32)
                        + rbufL_h2[c].astype(jnp.float32))
                    fin_h2[c] = out_ref[rows(1, c), :].astype(jnp.bfloat16)
                    r = copy(fin_h2.at[c], fin_h2.at[c],
                             s_finA_h2.at[c], finr_h2.at[c], 3)
                    r.start(); sends.append(r)
                    r = copy(fin_h2.at[c], fin_h2.at[c],
                             s_finB_h2.at[c], finr_h2.at[c], 1)
                    r.start(); sends.append(r)
                    copy(fin_h1.at[c], fin_h1.at[c],
                         s_finB_h1.at[c], finr_h1.at[c], 1).wait_recv()
                    out_ref[rows(0, c), :] = fin_h1[c].astype(jnp.float32)
                    r = copy(fin_h1.at[c], fin_h1.at[c],
                             s_finA_h1.at[c], finr_h1.at[c], 3)
                    r.start(); sends.append(r)
                for r in sends:
                    r.wait_send()

    chunk_buf = pltpu.VMEM((NC, CR, d), jnp.bfloat16)
    sem = pltpu.SemaphoreType.DMA((NC,))
    return pl.pallas_call(
        body,
        grid=(nm,),
        in_specs=[
            pl.BlockSpec((M_BLK, k), lambda i: (i, 0)),
            pl.BlockSpec((d, k), lambda i: (0, 0)),
        ],
        out_specs=pl.BlockSpec((m, d), lambda i: (0, 0)),
        out_shape=jax.ShapeDtypeStruct((m, d), jnp.float32),
        scratch_shapes=[
            pltpu.VMEM((d, k), jnp.bfloat16),
            chunk_buf, chunk_buf,
            chunk_buf, chunk_buf,
            chunk_buf, chunk_buf,
            chunk_buf, chunk_buf,
            sem, sem, sem, sem, sem, sem,
            sem, sem, sem, sem, sem, sem,
        ],
        compiler_params=pltpu.CompilerParams(
            collective_id=0,
            dimension_semantics=("arbitrary",),
            vmem_limit_bytes=60 * 1024 * 1024,
        ),
    )(dy, W)


# device time: 90401 ns/iter; 1.1040x vs baseline; 1.1040x over previous
import jax
import jax.numpy as jnp
from jax import lax
from jax.experimental import pallas as pl
from jax.experimental.pallas import tpu as pltpu

Z = 4
M_BLK = 128
NC = 8
HALF = 512
CR = HALF // NC


def kernel(dy, W):
    m, k = dy.shape
    d = W.shape[0]
    nm = m // M_BLK

    def body(dy_ref, w_ref, out_ref, w_bf,
             sbuf_h1, sbuf_h2, rbufL_h1, rbufL_h2, rbufR_h1, rbufR_h2,
             fin_h1, fin_h2,
             rL_h1, rL_h2, rR_h1, rR_h2, finr_h1, finr_h2,
             s_red_h1, s_red_h2, s_finA_h1, s_finB_h1, s_finA_h2, s_finB_h2):
        i = pl.program_id(0)

        @pl.when(i == 0)
        def _():
            w_bf[...] = w_ref[...].astype(jnp.bfloat16)

        out_ref[pl.ds(i * M_BLK, M_BLK), :] = lax.dot_general(
            dy_ref[...].astype(jnp.bfloat16),
            w_bf[...],
            (((1,), (1,)), ((), ())),
            preferred_element_type=jnp.float32,
        )

        @pl.when(i == nm - 1)
        def _():
            my_x = lax.axis_index("x")
            my_y = lax.axis_index("y")
            my_z = lax.axis_index("z")

            def dev(zz):
                return dict(device_id=(my_x, my_y, zz),
                            device_id_type=pl.DeviceIdType.MESH)

            def copy(src, dst, ssem, rsem, zz):
                return pltpu.make_async_remote_copy(
                    src_ref=src, dst_ref=dst, send_sem=ssem, recv_sem=rsem,
                    **dev(zz))

            def rows(h, c):
                return pl.ds(h * HALF + c * CR, CR)

            bar = pltpu.get_barrier_semaphore()

            def barrier(nbrs):
                for nb in nbrs:
                    pl.semaphore_signal(bar, inc=1, **dev(nb))
                pl.semaphore_wait(bar, len(nbrs))

            @pl.when(my_z == 0)
            def _():
                barrier([1])
                sends = []
                for c in range(NC):
                    sbuf_h1[c] = out_ref[rows(0, c), :].astype(jnp.bfloat16)
                    r = copy(sbuf_h1.at[c], rbufL_h1.at[c],
                             s_red_h1.at[c], rL_h1.at[c], 1)
                    r.start(); sends.append(r)
                    sbuf_h2[c] = out_ref[rows(1, c), :].astype(jnp.bfloat16)
                    r = copy(sbuf_h2.at[c], rbufL_h2.at[c],
                             s_red_h2.at[c], rL_h2.at[c], 1)
                    r.start(); sends.append(r)
                for c in range(NC):
                    copy(fin_h1.at[c], fin_h1.at[c],
                         s_finA_h1.at[c], finr_h1.at[c], 1).wait_recv()
                    out_ref[rows(0, c), :] = fin_h1[c].astype(jnp.float32)
                    copy(fin_h2.at[c], fin_h2.at[c],
                         s_finA_h2.at[c], finr_h2.at[c], 1).wait_recv()
                    out_ref[rows(1, c), :] = fin_h2[c].astype(jnp.float32)
                for r in sends:
                    r.wait_send()

            @pl.when(my_z == 3)
            def _():
                barrier([2])
                sends = []
                for c in range(NC):
                    sbuf_h2[c] = out_ref[rows(1, c), :].astype(jnp.bfloat16)
                    r = copy(sbuf_h2.at[c], rbufR_h2.at[c],
                             s_red_h2.at[c], rR_h2.at[c], 2)
                    r.start(); sends.append(r)
                    sbuf_h1[c] = out_ref[rows(0, c), :].astype(jnp.bfloat16)
                    r = copy(sbuf_h1.at[c], rbufR_h1.at[c],
                             s_red_h1.at[c], rR_h1.at[c], 2)
                    r.start(); sends.append(r)
                for c in range(NC):
                    copy(fin_h2.at[c], fin_h2.at[c],
                         s_finA_h2.at[c], finr_h2.at[c], 2).wait_recv()
                    out_ref[rows(1, c), :] = fin_h2[c].astype(jnp.float32)
                    copy(fin_h1.at[c], fin_h1.at[c],
                         s_finA_h1.at[c], finr_h1.at[c], 2).wait_recv()
                    out_ref[rows(0, c), :] = fin_h1[c].astype(jnp.float32)
                for r in sends:
                    r.wait_send()

            @pl.when(my_z == 1)
            def _():
                barrier([0, 2])
                sends = []
                for c in range(NC):
                    copy(rbufL_h2.at[c], rbufL_h2.at[c],
                         s_red_h2.at[c], rL_h2.at[c], 0).wait_recv()
                    sbuf_h2[c] = (rbufL_h2[c].astype(jnp.float32)
                                  + out_ref[rows(1, c), :]).astype(jnp.bfloat16)
                    r = copy(sbuf_h2.at[c], rbufL_h2.at[c],
                             s_red_h2.at[c], rL_h2.at[c], 2)
                    r.start(); sends.append(r)
                for c in range(NC):
                    copy(rbufL_h1.at[c], rbufL_h1.at[c],
                         s_red_h1.at[c], rL_h1.at[c], 0).wait_recv()
                    copy(rbufR_h1.at[c], rbufR_h1.at[c],
                         s_red_h1.at[c], rR_h1.at[c], 2).wait_recv()
                    out_ref[rows(0, c), :] += (
                        rbufL_h1[c].astype(jnp.float32)
                        + rbufR_h1[c].astype(jnp.float32))
                    fin_h1[c] = out_ref[rows(0, c), :].astype(jnp.bfloat16)
                    r = copy(fin_h1.at[c], fin_h1.at[c],
                             s_finA_h1.at[c], finr_h1.at[c], 0)
                    r.start(); sends.append(r)
                    r = copy(fin_h1.at[c], fin_h1.at[c],
                             s_finB_h1.at[c], finr_h1.at[c], 2)
                    r.start(); sends.append(r)
                for c in range(NC):
                    copy(fin_h2.at[c], fin_h2.at[c],
                         s_finB_h2.at[c], finr_h2.at[c], 2).wait_recv()
                    out_ref[rows(1, c), :] = fin_h2[c].astype(jnp.float32)
                    r = copy(fin_h2.at[c], fin_h2.at[c],
                             s_finA_h2.at[c], finr_h2.at[c], 0)
                    r.start(); sends.append(r)
                for r in sends:
                    r.wait_send()

            @pl.when(my_z == 2)
            def _():
                barrier([1, 3])
                sends = []
                for c in range(NC):
                    copy(rbufR_h1.at[c], rbufR_h1.at[c],
                         s_red_h1.at[c], rR_h1.at[c], 3).wait_recv()
                    sbuf_h1[c] = (rbufR_h1[c].astype(jnp.float32)
                                  + out_ref[rows(0, c), :]).astype(jnp.bfloat16)
                    r = copy(sbuf_h1.at[c], rbufR_h1.at[c],
                             s_red_h1.at[c], rR_h1.at[c], 1)
                    r.start(); sends.append(r)
                for c in range(NC):
                    copy(rbufR_h2.at[c], rbufR_h2.at[c],
                         s_red_h2.at[c], rR_h2.at[c], 3).wait_recv()
                    copy(rbufL_h2.at[c], rbufL_h2.at[c],
                         s_red_h2.at[c], rL_h2.at[c], 1).wait_recv()
                    out_ref[rows(1, c), :] += (
                        rbufR_h2[c].astype(jnp.float32)
                        + rbufL_h2[c].astype(jnp.float32))
                    fin_h2[c] = out_ref[rows(1, c), :].astype(jnp.bfloat16)
                    r = copy(fin_h2.at[c], fin_h2.at[c],
                             s_finA_h2.at[c], finr_h2.at[c], 3)
                    r.start(); sends.append(r)
                    r = copy(fin_h2.at[c], fin_h2.at[c],
                             s_finB_h2.at[c], finr_h2.at[c], 1)
                    r.start(); sends.append(r)
                for c in range(NC):
                    copy(fin_h1.at[c], fin_h1.at[c],
                         s_finB_h1.at[c], finr_h1.at[c], 1).wait_recv()
                    out_ref[rows(0, c), :] = fin_h1[c].astype(jnp.float32)
                    r = copy(fin_h1.at[c], fin_h1.at[c],
                             s_finA_h1.at[c], finr_h1.at[c], 3)
                    r.start(); sends.append(r)
                for r in sends:
                    r.wait_send()

    chunk_buf = pltpu.VMEM((NC, CR, d), jnp.bfloat16)
    sem = pltpu.SemaphoreType.DMA((NC,))
    return pl.pallas_call(
        body,
        grid=(nm,),
        in_specs=[
            pl.BlockSpec((M_BLK, k), lambda i: (i, 0)),
            pl.BlockSpec((d, k), lambda i: (0, 0)),
        ],
        out_specs=pl.BlockSpec((m, d), lambda i: (0, 0)),
        out_shape=jax.ShapeDtypeStruct((m, d), jnp.float32),
        scratch_shapes=[
            pltpu.VMEM((d, k), jnp.bfloat16),
            chunk_buf, chunk_buf,
            chunk_buf, chunk_buf,
            chunk_buf, chunk_buf,
            chunk_buf, chunk_buf,
            sem, sem, sem, sem, sem, sem,
            sem, sem, sem, sem, sem, sem,
        ],
        compiler_params=pltpu.CompilerParams(
            collective_id=0,
            dimension_semantics=("arbitrary",),
            vmem_limit_bytes=60 * 1024 * 1024,
        ),
    )(dy, W)


# device time: 55509 ns/iter; 1.7980x vs baseline; 1.6286x over previous
import jax
import jax.numpy as jnp
from jax import lax
from jax.experimental import pallas as pl
from jax.experimental.pallas import tpu as pltpu

M_BLK = 256
CR = 128


def kernel(dy, W):
    m, k = dy.shape
    d = W.shape[0]
    nm = m // M_BLK

    def body(dy_ref, w_ref, out_ref, w_f32, w_bfT,
             s1sbuf, s2sbuf, finkbuf, s1rbuf, s2rbuf, s3rbuf, xbuf, wsem,
             s1r, s2r, s3r, xr, s1s, s2s, s3s, xsk, xss):
        i = pl.program_id(0)

        my_x = lax.axis_index("x")
        my_y = lax.axis_index("y")
        my_z = lax.axis_index("z")
        za = my_z % 2

        def rdma(src, dst, ssem, rsem, did):
            return pltpu.make_async_remote_copy(
                src_ref=src, dst_ref=dst, send_sem=ssem, recv_sem=rsem,
                device_id=did, device_id_type=pl.DeviceIdType.MESH)

        p1 = (my_x, my_y, my_z ^ 1)
        p2 = (my_x, my_y, my_z ^ 2)
        px = (1 - my_x, my_y, my_z)

        def rowsj(jj):
            return pl.ds((2 * jj + my_x) * CR, CR)

        def rows_peer(jj):
            return pl.ds((2 * jj + (1 - my_x)) * CR, CR)

        @pl.when(i == 0)
        def _():
            cp = pltpu.make_async_copy(w_ref, w_f32, wsem)
            cp.start()
            bar = pltpu.get_barrier_semaphore()
            for did in (p1, p2, px):
                pl.semaphore_signal(bar, inc=1, device_id=did,
                                    device_id_type=pl.DeviceIdType.MESH)
            pl.semaphore_wait(bar, 3)
            cp.wait()
            w_bfT[...] = w_f32[...].astype(jnp.bfloat16).T

        out_ref[pl.ds(i * M_BLK, M_BLK), :] = lax.dot_general(
            dy_ref[...].astype(jnp.bfloat16),
            w_bfT[...],
            (((1,), (0,)), ((), ())),
            preferred_element_type=jnp.float32,
        )

        for X in range(nm):
            @pl.when(i == X)
            def _(X=X):
                for t in range(2):
                    @pl.when((1 - za + 2 * t) == X)
                    def _(t=t):
                        jj = 1 - za + 2 * t
                        s1sbuf[t] = out_ref[rowsj(jj), :].astype(jnp.bfloat16)
                        rdma(s1sbuf.at[t], s1rbuf.at[t],
                             s1s.at[t], s1r.at[t], p1).start()

                    @pl.when(jnp.minimum(za + 2 * t + 1, nm - 1) == X)
                    def _(t=t):
                        jj = za + 2 * t
                        rdma(s1rbuf.at[t], s1rbuf.at[t],
                             s1s.at[t], s1r.at[t], p1).wait_recv()
                        out_ref[rowsj(jj), :] += s1rbuf[t].astype(jnp.float32)
                        s2sbuf[t] = out_ref[rowsj(jj), :].astype(jnp.bfloat16)
                        rdma(s2sbuf.at[t], s2rbuf.at[t],
                             s2s.at[t], s2r.at[t], p2).start()

                    @pl.when(jnp.minimum(za + 2 * t + 2, nm - 1) == X)
                    def _(t=t):
                        jj = za + 2 * t
                        rdma(s2rbuf.at[t], s2rbuf.at[t],
                             s2s.at[t], s2r.at[t], p2).wait_recv()
                        out_ref[rowsj(jj), :] += s2rbuf[t].astype(jnp.float32)
                        finkbuf[t] = out_ref[rowsj(jj), :].astype(jnp.bfloat16)
                        rdma(finkbuf.at[t], s3rbuf.at[t],
                             s3s.at[t], s3r.at[t], p1).start()
                        rdma(finkbuf.at[t], xbuf.at[t],
                             xsk.at[t], xr.at[t], px).start()

        @pl.when(i == nm - 1)
        def _():
            for t in range(2):
                jj = 1 - za + 2 * t
                rdma(s3rbuf.at[t], s3rbuf.at[t],
                     s3s.at[t], s3r.at[t], p1).wait_recv()
                out_ref[rowsj(jj), :] = s3rbuf[t].astype(jnp.float32)
                rdma(s3rbuf.at[t], xbuf.at[2 + t],
                     xss.at[t], xr.at[2 + t], px).start()
            for t in range(2):
                rdma(xbuf.at[t], xbuf.at[t], xsk.at[t], xr.at[t],
                     px).wait_recv()
                out_ref[rows_peer(za + 2 * t), :] = (
                    xbuf[t].astype(jnp.float32))
                rdma(xbuf.at[2 + t], xbuf.at[2 + t], xss.at[t], xr.at[2 + t],
                     px).wait_recv()
                out_ref[rows_peer(1 - za + 2 * t), :] = (
                    xbuf[2 + t].astype(jnp.float32))
            for t in range(2):
                rdma(s1sbuf.at[t], s1rbuf.at[t], s1s.at[t], s1r.at[t],
                     p1).wait_send()
                rdma(s2sbuf.at[t], s2rbuf.at[t], s2s.at[t], s2r.at[t],
                     p2).wait_send()
                rdma(finkbuf.at[t], s3rbuf.at[t], s3s.at[t], s3r.at[t],
                     p1).wait_send()
                rdma(finkbuf.at[t], xbuf.at[t], xsk.at[t], xr.at[t],
                     px).wait_send()
                rdma(s3rbuf.at[t], xbuf.at[2 + t], xss.at[t], xr.at[2 + t],
                     px).wait_send()

    buf2 = pltpu.VMEM((2, CR, d), jnp.bfloat16)
    sem2 = pltpu.SemaphoreType.DMA((2,))
    return pl.pallas_call(
        body,
        grid=(nm,),
        in_specs=[
            pl.BlockSpec((M_BLK, k), lambda i: (i, 0)),
            pl.BlockSpec(memory_space=pltpu.HBM),
        ],
        out_specs=pl.BlockSpec((m, d), lambda i: (0, 0)),
        out_shape=jax.ShapeDtypeStruct((m, d), jnp.float32),
        scratch_shapes=[
            pltpu.VMEM((d, k), jnp.float32),
            pltpu.VMEM((k, d), jnp.bfloat16),
            buf2, buf2, buf2,
            buf2, buf2, buf2,
            pltpu.VMEM((4, CR, d), jnp.bfloat16),
            pltpu.SemaphoreType.DMA,
            sem2, sem2, sem2,
            pltpu.SemaphoreType.DMA((4,)),
            sem2, sem2, sem2, sem2, sem2,
        ],
        compiler_params=pltpu.CompilerParams(
            collective_id=0,
            dimension_semantics=("arbitrary",),
            vmem_limit_bytes=60 * 1024 * 1024,
        ),
    )(dy, W)
